# Initial kernel scaffold; baseline (speedup 1.0000x reference)
import jax
import jax.numpy as jnp
from jax.experimental import pallas as pl


def kernel(hidden_states, W1, b1, W2, b2):
    raise NotImplementedError("write your pallas kernel here")


# speedup vs baseline: 3.1248x; 3.1248x over previous
import functools

import jax
import jax.numpy as jnp
from jax.experimental import pallas as pl
from jax.experimental.pallas import tpu as pltpu

B = 4
S = 4096
HIDDEN = 2048
H4 = HIDDEN // 4
K = S // 2
TILE = 1024
NTILES = (B * S) // TILE


def _mlp_body(x_ref, w1_ref, b1_ref, w2_ref, b2_ref, scores_ref):
    x = x_ref[...]
    h = jnp.dot(x, w1_ref[...], preferred_element_type=jnp.float32) + b1_ref[...]
    g = h * (0.5 * (jax.lax.erf(h * jnp.float32(0.7071067811865476)) + 1.0))
    logits = jnp.dot(g, w2_ref[...], preferred_element_type=jnp.float32) + b2_ref[...]
    scores_ref[...] = jax.nn.sigmoid(logits)


def _select_body(scores_ref, w_ref, m_ref):
    s = scores_ref[...]
    key = jax.lax.bitcast_convert_type(s, jnp.int32)

    def step(i, p):
        b = 30 - i
        q = p | (1 << b)
        c = jnp.sum(((key >> b) >= (q >> b)).astype(jnp.int32), axis=1,
                    keepdims=True)
        return jnp.where(c >= K, q, p)

    p = jax.lax.fori_loop(0, 31, step, jnp.zeros((B, 1), jnp.int32))

    gt = key > p
    eq = key == p
    need = K - jnp.sum(gt.astype(jnp.int32), axis=1, keepdims=True)

    lo = (S - 1) - jax.lax.broadcasted_iota(jnp.int32, (B, S), 1)

    def step2(i, plo):
        b = 11 - i
        q = plo | (1 << b)
        c = jnp.sum((eq & ((lo >> b) >= (q >> b))).astype(jnp.int32), axis=1,
                    keepdims=True)
        return jnp.where(c >= need, q, plo)

    plo = jax.lax.fori_loop(0, 12, step2, jnp.zeros((B, 1), jnp.int32))

    mask = gt | (eq & (lo >= plo))
    m_ref[...] = mask
    w_ref[...] = s * mask.astype(s.dtype)


@jax.jit
def kernel(hidden_states, W1, b1, W2, b2):
    x = hidden_states.reshape(B * S, HIDDEN)
    scores = pl.pallas_call(
        _mlp_body,
        grid=(NTILES,),
        in_specs=[
            pl.BlockSpec((TILE, HIDDEN), lambda i: (i, 0)),
            pl.BlockSpec((HIDDEN, H4), lambda i: (0, 0)),
            pl.BlockSpec((1, H4), lambda i: (0, 0)),
            pl.BlockSpec((H4, 1), lambda i: (0, 0)),
            pl.BlockSpec((1, 1), lambda i: (0, 0)),
        ],
        out_specs=pl.BlockSpec((TILE, 1), lambda i: (i, 0)),
        out_shape=jax.ShapeDtypeStruct((B * S, 1), jnp.float32),
        compiler_params=pltpu.CompilerParams(
            dimension_semantics=("arbitrary",)),
    )(x, W1, b1.reshape(1, H4), W2, b2.reshape(1, 1))
    scores = scores.reshape(B, S)

    weights, mask = pl.pallas_call(
        _select_body,
        out_shape=(
            jax.ShapeDtypeStruct((B, S), jnp.float32),
            jax.ShapeDtypeStruct((B, S), jnp.bool_),
        ),
    )(scores)
    return weights, mask
